# Initial kernel scaffold; baseline (speedup 1.0000x reference)
import jax
import jax.numpy as jnp
from jax.experimental import pallas as pl


def kernel(p1, x1, o1, p2, x2, o2, W, b, gamma, beta):
    raise NotImplementedError("write your pallas kernel here")


# speedup vs baseline: 11.3040x; 11.3040x over previous
import functools

import jax
import jax.numpy as jnp
from jax import lax
from jax.experimental import pallas as pl
from jax.experimental.pallas import tpu as pltpu
from jax.experimental.pallas import tpu_sc as plsc

B = 4
N1 = 65536
N2 = 16384
n1 = N1 // B
n2 = N2 // B
D = 64
K = 3
T1 = 256
T2 = 512

_NC = 2
_NS = 16
_NW = _NC * _NS
_ROWS = N1 * K
_CH = 128
_NCH = _ROWS // (_NW * _CH)


def _knn_body(p1_ref, p2t_ref, idx_ref, w_ref):
    g = pl.program_id(0)
    b = g // (n1 // T1)
    pts = p1_ref[...]
    p2t = p2t_ref[0]
    dx = pts[:, 0:1] - p2t[0:1, :]
    dy = pts[:, 1:2] - p2t[1:2, :]
    dz = pts[:, 2:3] - p2t[2:3, :]
    d2 = dx * dx + dy * dy + dz * dz
    col = lax.broadcasted_iota(jnp.int32, (T1, n2), 1)
    js = []
    ws = []
    for _ in range(K):
        m = jnp.min(d2, axis=1, keepdims=True)
        j = jnp.min(jnp.where(d2 == m, col, n2), axis=1, keepdims=True)
        d2 = jnp.where(col == j, jnp.float32(1e30), d2)
        ws.append(1.0 / (jnp.sqrt(m) + 1e-8))
        js.append(j)
    wsum = ws[0] + ws[1] + ws[2]
    w_ref[...] = jnp.concatenate(ws, axis=1) / wsum
    idx_ref[...] = jnp.concatenate(js, axis=1) + b * n2


@functools.lru_cache(maxsize=1)
def _make_sc_gather():
    return functools.partial(
        pl.kernel,
        mesh=plsc.VectorSubcoreMesh(core_axis_name="c", subcore_axis_name="s"),
        compiler_params=pltpu.CompilerParams(use_tc_tiling_on_sc=False),
        out_type=jax.ShapeDtypeStruct((_ROWS, D), jnp.float32),
        scratch_types=[
            pltpu.VMEM((_NCH, _CH), jnp.int32),
            pltpu.VMEM((_CH, D), jnp.float32),
            pltpu.VMEM((_CH, D), jnp.float32),
            pltpu.SemaphoreType.DMA,
            pltpu.SemaphoreType.DMA,
        ],
    )(_sc_gather_body)


def _sc_gather_body(x2_hbm, idx_hbm, out_hbm, idx_v, buf0, buf1, sem0, sem1):
    wid = lax.axis_index("s") * _NC + lax.axis_index("c")
    pltpu.sync_copy(idx_hbm.at[pl.ds(wid * _NCH, _NCH)], idx_v)
    base = wid * _NCH * _CH

    def body(jj, carry):
        j0 = jj * 2
        cp0 = pltpu.make_async_copy(x2_hbm.at[idx_v.at[j0]], buf0, sem0)
        cp0.start()
        cp1 = pltpu.make_async_copy(x2_hbm.at[idx_v.at[j0 + 1]], buf1, sem1)
        cp1.start()
        cp0.wait()
        pltpu.sync_copy(buf0, out_hbm.at[pl.ds(base + j0 * _CH, _CH)])
        cp1.wait()
        pltpu.sync_copy(buf1, out_hbm.at[pl.ds(base + (j0 + 1) * _CH, _CH)])
        return carry

    lax.fori_loop(0, _NCH // 2, body, 0)


def _mlp_body(x1_ref, f0_ref, f1_ref, f2_ref, w_ref, w1_ref, w2_ref, b_ref,
              h_ref, s_ref, ss_ref):
    g = pl.program_id(0)
    w = w_ref[...]
    interp = (f0_ref[...] * w[:, 0:1] + f1_ref[...] * w[:, 1:2]
              + f2_ref[...] * w[:, 2:3])
    h = (jnp.dot(x1_ref[...], w1_ref[...], preferred_element_type=jnp.float32,
                 precision=lax.Precision.HIGHEST)
         + jnp.dot(interp, w2_ref[...], preferred_element_type=jnp.float32,
                   precision=lax.Precision.HIGHEST)
         + b_ref[...])
    h_ref[...] = h
    hr = h.reshape(8, T2 // 8, D)
    ps = jnp.sum(hr, axis=1)
    pss = jnp.sum(hr * hr, axis=1)

    @pl.when(g == 0)
    def _():
        s_ref[...] = ps
        ss_ref[...] = pss

    @pl.when(g > 0)
    def _():
        s_ref[...] = s_ref[...] + ps
        ss_ref[...] = ss_ref[...] + pss


def _bn_body(h_ref, s_ref, ss_ref, gm_ref, bt_ref, o_ref):
    s = jnp.sum(s_ref[...], axis=0, keepdims=True)
    ss = jnp.sum(ss_ref[...], axis=0, keepdims=True)
    mu = s / N1
    var = ss / N1 - mu * mu
    inv = lax.rsqrt(var + 1e-5)
    h = h_ref[...]
    o_ref[...] = jnp.maximum(gm_ref[...] * ((h - mu) * inv) + bt_ref[...], 0.0)


def kernel(p1, x1, o1, p2, x2, o2, W, b, gamma, beta):
    p2t = p2.reshape(B, n2, 3).transpose(0, 2, 1)
    nt1 = n1 // T1
    idx, w = pl.pallas_call(
        _knn_body,
        grid=(B * nt1,),
        in_specs=[
            pl.BlockSpec((T1, 3), lambda g: (g, 0)),
            pl.BlockSpec((1, 3, n2), lambda g: (g // nt1, 0, 0)),
        ],
        out_specs=[
            pl.BlockSpec((T1, K), lambda g: (g, 0)),
            pl.BlockSpec((T1, K), lambda g: (g, 0)),
        ],
        out_shape=[
            jax.ShapeDtypeStruct((N1, K), jnp.int32),
            jax.ShapeDtypeStruct((N1, K), jnp.float32),
        ],
    )(p1, p2t)

    idx_km = idx.T.reshape(_ROWS // _CH, _CH)
    feats = _make_sc_gather()(x2, idx_km)
    f0 = feats[0:N1]
    f1 = feats[N1:2 * N1]
    f2 = feats[2 * N1:3 * N1]

    nt2 = N1 // T2
    row_spec = pl.BlockSpec((T2, D), lambda g: (g, 0))
    full64 = pl.BlockSpec((D, D), lambda g: (0, 0))
    stat_spec = pl.BlockSpec((8, D), lambda g: (0, 0))
    h, s, ss = pl.pallas_call(
        _mlp_body,
        grid=(nt2,),
        in_specs=[
            row_spec, row_spec, row_spec, row_spec,
            pl.BlockSpec((T2, K), lambda g: (g, 0)),
            full64, full64,
            pl.BlockSpec((1, D), lambda g: (0, 0)),
        ],
        out_specs=[row_spec, stat_spec, stat_spec],
        out_shape=[
            jax.ShapeDtypeStruct((N1, D), jnp.float32),
            jax.ShapeDtypeStruct((8, D), jnp.float32),
            jax.ShapeDtypeStruct((8, D), jnp.float32),
        ],
    )(x1, f0, f1, f2, w, W[:D], W[D:], b.reshape(1, D))

    x = pl.pallas_call(
        _bn_body,
        grid=(nt2,),
        in_specs=[
            row_spec, stat_spec, stat_spec,
            pl.BlockSpec((1, D), lambda g: (0, 0)),
            pl.BlockSpec((1, D), lambda g: (0, 0)),
        ],
        out_specs=row_spec,
        out_shape=jax.ShapeDtypeStruct((N1, D), jnp.float32),
    )(h, s, ss, gamma.reshape(1, D), beta.reshape(1, D))

    return (p1, x, o1)
